# Initial kernel scaffold; baseline (speedup 1.0000x reference)
import jax
import jax.numpy as jnp
from jax.experimental import pallas as pl


def kernel(nuclei, params, charges):
    raise NotImplementedError("write your pallas kernel here")


# speedup vs baseline: 6.1939x; 6.1939x over previous
import functools
import math

import jax
import jax.numpy as jnp
from jax import lax
from jax.experimental import pallas as pl

_N = 256
_SB = 32
_CUTOFF = 10.0
_COEF = math.sqrt(2.0 / _CUTOFF)


def _tc_body(nuc_ref, ch_ref, f_ref, emb_ref,
             m0s_ref, m0r_ref, m0e_ref, m0b0_ref, m0w1_ref, m0b1_ref,
             u0n_ref, u0m_ref, u0b0_ref, u0w1_ref, u0b1_ref,
             m1s_ref, m1r_ref, m1e_ref, m1b0_ref, m1w1_ref, m1b1_ref,
             u1n_ref, u1m_ref, u1b0_ref, u1w1_ref, u1b1_ref,
             n0w0_ref, n0b0_ref, n0w1_ref, ne0_ref,
             n1w0_ref, n1b0_ref, n1w1_ref, ne1_ref,
             gw0_ref, gb0_ref, gw1_ref, gb1_ref,
             o0_ref, o1_ref, og_ref):
    nuc = nuc_ref[...]
    ch = ch_ref[...]
    f = f_ref[...]

    onehot = (lax.broadcasted_iota(jnp.int32, (_N, 8), 1) == ch)
    onehot = onehot.astype(jnp.float32)
    n0 = jnp.dot(onehot, emb_ref[...])

    f3 = (f * (1.0 / _CUTOFF))[None]

    def edge_phase(n, ws, wr, we, b0):
        A = jnp.dot(n, ws)
        B = jnp.dot(n, wr)
        do = ws.shape[1]
        chunks = []
        for c in range(_N // _SB):
            s0 = c * _SB
            d2 = None
            for ax in range(3):
                xs = nuc[s0:s0 + _SB, ax:ax + 1][:, :, None]
                xr = nuc[:, ax:ax + 1][None, :, :]
                dx = xs - xr
                d2 = dx * dx if d2 is None else d2 + dx * dx
            xe = jnp.sqrt(d2) + 1e-8
            arg = xe * f3
            e3 = jnp.sin(arg) * (_COEF / xe)
            ep = jnp.dot(e3.reshape(_SB * _N, 32), we)
            pre = (ep.reshape(_SB, _N, do)
                   + A[s0:s0 + _SB][:, None, :]
                   + B[None, :, :]
                   + b0[None])
            sil = pre * jax.nn.sigmoid(pre)
            sid = lax.broadcasted_iota(jnp.int32, (_SB, _N, 1), 0) + s0
            rid = lax.broadcasted_iota(jnp.int32, (_SB, _N, 1), 1)
            sil = jnp.where(sid == rid, 0.0, sil)
            chunks.append(jnp.sum(sil, axis=1))
        return jnp.concatenate(chunks, axis=0)

    def silu(x):
        return x * jax.nn.sigmoid(x)

    ms0 = edge_phase(n0, m0s_ref[...], m0r_ref[...], m0e_ref[...],
                     m0b0_ref[...])
    msg0 = jnp.dot(ms0 / 255.0, m0w1_ref[...]) + m0b1_ref[...]
    t0 = jnp.dot(n0, u0n_ref[...]) + jnp.dot(msg0, u0m_ref[...]) + u0b0_ref[...]
    n1 = jnp.dot(silu(t0), u0w1_ref[...]) + u0b1_ref[...]

    ms1 = edge_phase(n1, m1s_ref[...], m1r_ref[...], m1e_ref[...],
                     m1b0_ref[...])
    msg1 = jnp.dot(ms1 / 255.0, m1w1_ref[...]) + m1b1_ref[...]
    t1 = jnp.dot(n1, u1n_ref[...]) + jnp.dot(msg1, u1m_ref[...]) + u1b0_ref[...]
    n2 = n1 + (jnp.dot(silu(t1), u1w1_ref[...]) + u1b1_ref[...])

    h = jnp.concatenate([n0, n1, n2], axis=1)

    o0 = (jnp.dot(silu(jnp.dot(h, n0w0_ref[...]) + n0b0_ref[...]),
                  n0w1_ref[...])
          + jnp.dot(onehot, ne0_ref[...]))
    o1 = (jnp.dot(silu(jnp.dot(h, n1w0_ref[...]) + n1b0_ref[...]),
                  n1w1_ref[...])
          + jnp.dot(onehot, ne1_ref[...]))
    gi = jnp.mean(h, axis=0, keepdims=True)
    og = (jnp.dot(silu(jnp.dot(gi, gw0_ref[...]) + gb0_ref[...]),
                  gw1_ref[...])
          + gb1_ref[...])

    o0_ref[...] = o0
    o1_ref[...] = o1
    og_ref[...] = og


@functools.partial(jax.jit, static_argnames=())
def kernel(nuclei, params, charges):
    p = params
    m0W0 = p['mp0']['W0']
    m1W0 = p['mp1']['W0']
    u0W0 = p['up0']['W0']
    u1W0 = p['up1']['W0']

    def row(b):
        return b.reshape(1, -1)

    args = (
        nuclei.reshape(_N, 3),
        jnp.clip(charges.reshape(_N, 1).astype(jnp.int32), 0, 7),
        row(p['rbf_f']),
        p['embed'],
        m0W0[0:32], m0W0[32:64], m0W0[64:96], row(p['mp0']['b0']),
        p['mp0']['W1'], row(p['mp0']['b1']),
        u0W0[0:32], u0W0[32:64], row(p['up0']['b0']),
        p['up0']['W1'], row(p['up0']['b1']),
        m1W0[0:64], m1W0[64:128], m1W0[128:160], row(p['mp1']['b0']),
        p['mp1']['W1'], row(p['mp1']['b1']),
        u1W0[0:64], u1W0[64:96], row(p['up1']['b0']),
        p['up1']['W1'], row(p['up1']['b1']),
        p['node0_mlp']['W0'], row(p['node0_mlp']['b0']),
        p['node0_mlp']['W1'], p['node0_embed'],
        p['node1_mlp']['W0'], row(p['node1_mlp']['b0']),
        p['node1_mlp']['W1'], p['node1_embed'],
        p['glob0_mlp']['W0'], row(p['glob0_mlp']['b0']),
        p['glob0_mlp']['W1'], row(p['glob0_mlp']['b1']),
    )

    o0, o1, og = pl.pallas_call(
        _tc_body,
        out_shape=(
            jax.ShapeDtypeStruct((_N, 1), jnp.float32),
            jax.ShapeDtypeStruct((_N, 3), jnp.float32),
            jax.ShapeDtypeStruct((1, 1), jnp.float32),
        ),
    )(*args)
    return (o0, o1, og.reshape(1))


# speedup vs baseline: 7.2404x; 1.1689x over previous
import functools
import math

import jax
import jax.numpy as jnp
from jax import lax
from jax.experimental import pallas as pl

_N = 256
_SB = 32
_CUTOFF = 10.0
_COEF = math.sqrt(2.0 / _CUTOFF)


def _tc_body(nuc_ref, ch_ref, f_ref, emb_ref,
             m0s_ref, m0r_ref, m0e_ref, m0b0_ref, m0w1_ref, m0b1_ref,
             u0n_ref, u0m_ref, u0b0_ref, u0w1_ref, u0b1_ref,
             m1s_ref, m1r_ref, m1e_ref, m1b0_ref, m1w1_ref, m1b1_ref,
             u1n_ref, u1m_ref, u1b0_ref, u1w1_ref, u1b1_ref,
             n0w0_ref, n0b0_ref, n0w1_ref, ne0_ref,
             n1w0_ref, n1b0_ref, n1w1_ref, ne1_ref,
             gw0_ref, gb0_ref, gw1_ref, gb1_ref,
             o0_ref, o1_ref, og_ref):
    nuc = nuc_ref[...]
    ch = ch_ref[...]
    f = f_ref[...]

    onehot = (lax.broadcasted_iota(jnp.int32, (_N, 8), 1) == ch)
    onehot = onehot.astype(jnp.float32)
    n0 = jnp.dot(onehot, emb_ref[...])

    f3 = (f * (1.0 / _CUTOFF))[None]
    n_ch = _N // _SB

    def edge_phase(n, ws, wr, we, b0):
        A = jnp.dot(n, ws)
        B = jnp.dot(n, wr)
        do = ws.shape[1]
        chunks = []
        for c in range(n_ch):
            s0 = c * _SB
            d2 = None
            for ax in range(3):
                xs = nuc[s0:s0 + _SB, ax:ax + 1]
                xr = nuc[:, ax:ax + 1].reshape(1, _N)
                dx = xs - xr
                d2 = dx * dx if d2 is None else d2 + dx * dx
            xe2 = jnp.sqrt(d2) + 1e-8
            inv2 = _COEF / xe2
            xe3 = xe2.reshape(_SB, _N, 1)
            inv3 = inv2.reshape(_SB, _N, 1)
            e3 = jnp.sin(xe3 * f3) * inv3
            ep = jnp.dot(e3.reshape(_SB * _N, 32), we)
            pre = (ep.reshape(_SB, _N, do)
                   + A[s0:s0 + _SB][:, None, :]
                   + B[None, :, :]
                   + b0[None])
            sil = pre * jax.nn.sigmoid(pre)
            sid = lax.broadcasted_iota(jnp.int32, (_SB, _N, 1), 0) + s0
            rid = lax.broadcasted_iota(jnp.int32, (_SB, _N, 1), 1)
            sil = jnp.where(sid == rid, 0.0, sil)
            chunks.append(jnp.sum(sil, axis=1))
        return jnp.concatenate(chunks, axis=0)

    def silu(x):
        return x * jax.nn.sigmoid(x)

    ms0 = edge_phase(n0, m0s_ref[...], m0r_ref[...], m0e_ref[...],
                     m0b0_ref[...])
    msg0 = jnp.dot(ms0 / 255.0, m0w1_ref[...]) + m0b1_ref[...]
    t0 = jnp.dot(n0, u0n_ref[...]) + jnp.dot(msg0, u0m_ref[...]) + u0b0_ref[...]
    n1 = jnp.dot(silu(t0), u0w1_ref[...]) + u0b1_ref[...]

    ms1 = edge_phase(n1, m1s_ref[...], m1r_ref[...], m1e_ref[...],
                     m1b0_ref[...])
    msg1 = jnp.dot(ms1 / 255.0, m1w1_ref[...]) + m1b1_ref[...]
    t1 = jnp.dot(n1, u1n_ref[...]) + jnp.dot(msg1, u1m_ref[...]) + u1b0_ref[...]
    n2 = n1 + (jnp.dot(silu(t1), u1w1_ref[...]) + u1b1_ref[...])

    h = jnp.concatenate([n0, n1, n2], axis=1)

    o0 = (jnp.dot(silu(jnp.dot(h, n0w0_ref[...]) + n0b0_ref[...]),
                  n0w1_ref[...])
          + jnp.dot(onehot, ne0_ref[...]))
    o1 = (jnp.dot(silu(jnp.dot(h, n1w0_ref[...]) + n1b0_ref[...]),
                  n1w1_ref[...])
          + jnp.dot(onehot, ne1_ref[...]))
    gi = jnp.mean(h, axis=0, keepdims=True)
    og = (jnp.dot(silu(jnp.dot(gi, gw0_ref[...]) + gb0_ref[...]),
                  gw1_ref[...])
          + gb1_ref[...])

    o0_ref[...] = o0
    o1_ref[...] = o1
    og_ref[...] = og


@functools.partial(jax.jit, static_argnames=())
def kernel(nuclei, params, charges):
    p = params
    m0W0 = p['mp0']['W0']
    m1W0 = p['mp1']['W0']
    u0W0 = p['up0']['W0']
    u1W0 = p['up1']['W0']

    def row(b):
        return b.reshape(1, -1)

    args = (
        nuclei.reshape(_N, 3),
        jnp.clip(charges.reshape(_N, 1).astype(jnp.int32), 0, 7),
        row(p['rbf_f']),
        p['embed'],
        m0W0[0:32], m0W0[32:64], m0W0[64:96], row(p['mp0']['b0']),
        p['mp0']['W1'], row(p['mp0']['b1']),
        u0W0[0:32], u0W0[32:64], row(p['up0']['b0']),
        p['up0']['W1'], row(p['up0']['b1']),
        m1W0[0:64], m1W0[64:128], m1W0[128:160], row(p['mp1']['b0']),
        p['mp1']['W1'], row(p['mp1']['b1']),
        u1W0[0:64], u1W0[64:96], row(p['up1']['b0']),
        p['up1']['W1'], row(p['up1']['b1']),
        p['node0_mlp']['W0'], row(p['node0_mlp']['b0']),
        p['node0_mlp']['W1'], p['node0_embed'],
        p['node1_mlp']['W0'], row(p['node1_mlp']['b0']),
        p['node1_mlp']['W1'], p['node1_embed'],
        p['glob0_mlp']['W0'], row(p['glob0_mlp']['b0']),
        p['glob0_mlp']['W1'], row(p['glob0_mlp']['b1']),
    )

    o0, o1, og = pl.pallas_call(
        _tc_body,
        out_shape=(
            jax.ShapeDtypeStruct((_N, 1), jnp.float32),
            jax.ShapeDtypeStruct((_N, 3), jnp.float32),
            jax.ShapeDtypeStruct((1, 1), jnp.float32),
        ),
    )(*args)
    return (o0, o1, og.reshape(1))


# speedup vs baseline: 16.3297x; 2.2554x over previous
import functools
import math

import jax
import jax.numpy as jnp
from jax import lax
from jax.experimental import pallas as pl

_N = 256
_SB = 64
_G = 4
_CUTOFF = 10.0
_COEF = math.sqrt(2.0 / _CUTOFF)
_EPS = 1e-8


def _tc_body(nuc_ref, ch_ref, f_ref, kf_ref, emb_ref,
             bd0_ref, bd1_ref,
             m0s_ref, m0r_ref, m0e_ref, m0b0_ref, m0w1_ref, m0b1_ref,
             u0n_ref, u0m_ref, u0b0_ref, u0w1_ref, u0b1_ref,
             m1s_ref, m1r_ref, m1e_ref, m1b0_ref, m1w1_ref, m1b1_ref,
             u1n_ref, u1m_ref, u1b0_ref, u1w1_ref, u1b1_ref,
             n0w0_ref, n0b0_ref, n0w1_ref, ne0_ref,
             n1w0_ref, n1b0_ref, n1w1_ref, ne1_ref,
             gw0_ref, gb0_ref, gw1_ref, gb1_ref,
             o0_ref, o1_ref, og_ref):
    nuc = nuc_ref[...]
    ch = ch_ref[...]
    f = f_ref[...]
    f3 = (f * (1.0 / _CUTOFF))[None]
    kf3 = kf_ref[...][None]

    onehot = (lax.broadcasted_iota(jnp.int32, (_N, 8), 1) == ch)
    onehot = onehot.astype(jnp.float32)
    n0 = jnp.dot(onehot, emb_ref[...])

    def silu(x):
        return x * jax.nn.sigmoid(x)

    arg_blocks = []
    for g in range(_G):
        s0 = g * _SB
        d2 = None
        for ax in range(3):
            xs = nuc[s0:s0 + _SB, ax:ax + 1]
            xr = nuc[:, ax:ax + 1].reshape(1, _N)
            dx = xs - xr
            d2 = dx * dx if d2 is None else d2 + dx * dx
        xe3 = (jnp.sqrt(d2) + _EPS).reshape(_SB, _N, 1)
        arg_blocks.append(xe3 * f3)
    argc = jnp.concatenate(arg_blocks, axis=2)
    ec = (jnp.sin(argc) / argc) * kf3
    ef = ec.reshape(_SB * _N, _G * 32)

    ed = jnp.sin(f * (_EPS / _CUTOFF)) * (_COEF / _EPS)

    def layer(n, ws, wr, we, b0, bd, w1, b1, un, um, ub0, uw1, ub1):
        A = jnp.dot(n, ws) + b0
        B = jnp.dot(n, wr)
        A_cat = jnp.concatenate(
            [A[g * _SB:(g + 1) * _SB] for g in range(_G)], axis=1)
        B_cat = jnp.concatenate([B] * _G, axis=1)
        ep = jnp.dot(ef, bd).reshape(_SB, _N, _G * 32)
        pre = ep + A_cat[:, None, :] + B_cat[None]
        sil = pre * jax.nn.sigmoid(pre)
        mc = jnp.sum(sil, axis=1)
        msum = jnp.concatenate(
            [mc[:, g * 32:(g + 1) * 32] for g in range(_G)], axis=0)
        pre_d = A + B + jnp.dot(ed, we)
        msum = msum - silu(pre_d)
        msg = jnp.dot(msum / 255.0, w1) + b1
        t = jnp.dot(n, un) + jnp.dot(msg, um) + ub0
        return jnp.dot(silu(t), uw1) + ub1

    n1 = layer(n0, m0s_ref[...], m0r_ref[...], m0e_ref[...], m0b0_ref[...],
               bd0_ref[...], m0w1_ref[...], m0b1_ref[...],
               u0n_ref[...], u0m_ref[...], u0b0_ref[...],
               u0w1_ref[...], u0b1_ref[...])
    n2 = n1 + layer(n1, m1s_ref[...], m1r_ref[...], m1e_ref[...],
                    m1b0_ref[...], bd1_ref[...], m1w1_ref[...], m1b1_ref[...],
                    u1n_ref[...], u1m_ref[...], u1b0_ref[...],
                    u1w1_ref[...], u1b1_ref[...])

    h = jnp.concatenate([n0, n1, n2], axis=1)

    o0 = (jnp.dot(silu(jnp.dot(h, n0w0_ref[...]) + n0b0_ref[...]),
                  n0w1_ref[...])
          + jnp.dot(onehot, ne0_ref[...]))
    o1 = (jnp.dot(silu(jnp.dot(h, n1w0_ref[...]) + n1b0_ref[...]),
                  n1w1_ref[...])
          + jnp.dot(onehot, ne1_ref[...]))
    gi = jnp.mean(h, axis=0, keepdims=True)
    og = (jnp.dot(silu(jnp.dot(gi, gw0_ref[...]) + gb0_ref[...]),
                  gw1_ref[...])
          + gb1_ref[...])

    o0_ref[...] = o0
    o1_ref[...] = o1
    og_ref[...] = og


@functools.partial(jax.jit, static_argnames=())
def kernel(nuclei, params, charges):
    p = params
    m0W0 = p['mp0']['W0']
    m1W0 = p['mp1']['W0']
    u0W0 = p['up0']['W0']
    u1W0 = p['up1']['W0']

    def row(b):
        return b.reshape(1, -1)

    f_row = row(p['rbf_f'])
    kf_cat = jnp.tile(f_row * (_COEF / _CUTOFF), (1, _G))
    eye_g = jnp.eye(_G, dtype=jnp.float32)
    bd0 = jnp.kron(eye_g, m0W0[64:96])
    bd1 = jnp.kron(eye_g, m1W0[128:160])

    args = (
        nuclei.reshape(_N, 3),
        jnp.clip(charges.reshape(_N, 1).astype(jnp.int32), 0, 7),
        f_row, kf_cat,
        p['embed'],
        bd0, bd1,
        m0W0[0:32], m0W0[32:64], m0W0[64:96], row(p['mp0']['b0']),
        p['mp0']['W1'], row(p['mp0']['b1']),
        u0W0[0:32], u0W0[32:64], row(p['up0']['b0']),
        p['up0']['W1'], row(p['up0']['b1']),
        m1W0[0:64], m1W0[64:128], m1W0[128:160], row(p['mp1']['b0']),
        p['mp1']['W1'], row(p['mp1']['b1']),
        u1W0[0:64], u1W0[64:96], row(p['up1']['b0']),
        p['up1']['W1'], row(p['up1']['b1']),
        p['node0_mlp']['W0'], row(p['node0_mlp']['b0']),
        p['node0_mlp']['W1'], p['node0_embed'],
        p['node1_mlp']['W0'], row(p['node1_mlp']['b0']),
        p['node1_mlp']['W1'], p['node1_embed'],
        p['glob0_mlp']['W0'], row(p['glob0_mlp']['b0']),
        p['glob0_mlp']['W1'], row(p['glob0_mlp']['b1']),
    )

    o0, o1, og = pl.pallas_call(
        _tc_body,
        out_shape=(
            jax.ShapeDtypeStruct((_N, 1), jnp.float32),
            jax.ShapeDtypeStruct((_N, 3), jnp.float32),
            jax.ShapeDtypeStruct((1, 1), jnp.float32),
        ),
    )(*args)
    return (o0, o1, og.reshape(1))
